# baseline (speedup 1.0000x reference)
import functools

import jax
import jax.numpy as jnp
import numpy as np
from jax import lax
from jax.experimental import pallas as pl
from jax.experimental.pallas import tpu as pltpu
from jax.experimental.pallas import tpu_sc as plsc

_BINS = 30
_EPS = 1e-10
_ROWS_PER_BLOCK = 2048
_NSUB = 16
_SPLIT = 16384
_CHR = 32
_LOG2C = (7.55886e-09, 1.4426935, -0.72129303, 0.4801555, -0.35541746,
          0.26644915, -0.18072152, 0.09595723, -0.03323721, 0.005413892)
_LN2 = 0.6931471805599453
_NLL_EPS = 23.02585


def _edges_table() -> np.ndarray:
    e = np.arange(32, dtype=np.float32) / np.float32(_BINS)
    e[30] = np.float32(1.0) + np.float32(_EPS)
    e[31] = np.inf
    return e


def _rows_body(preds_ref, tgt_ref, g_ref, nll_ref):
    x = preds_ref[...]
    t = tgt_ref[0]
    m = jnp.max(x, axis=1, keepdims=True)
    e = jnp.exp(x - m)
    s = jnp.sum(e, axis=1, keepdims=True)
    col = lax.broadcasted_iota(jnp.int32, x.shape, 1)
    et = jnp.sum(jnp.where(col == t, e, 0.0), axis=1, keepdims=True)
    p = et / s
    pc = jnp.clip(p, jnp.float32(_EPS), jnp.float32(1.0 - _EPS))
    g_ref[0] = jnp.abs(pc - 1.0)
    nll_ref[0] = -jnp.log(pc)


def _row_stats(preds, targets):
    c = preds.shape[1]
    r = _ROWS_PER_BLOCK
    nb = _SPLIT // r
    g3, nll3 = pl.pallas_call(
        _rows_body,
        grid=(nb,),
        in_specs=[
            pl.BlockSpec((r, c), lambda i: (i, 0)),
            pl.BlockSpec((1, r, 1), lambda i: (i, 0, 0)),
        ],
        out_specs=[
            pl.BlockSpec((1, r, 1), lambda i: (i, 0, 0)),
            pl.BlockSpec((1, r, 1), lambda i: (i, 0, 0)),
        ],
        out_shape=[jax.ShapeDtypeStruct((nb, r, 1), jnp.float32)] * 2,
        compiler_params=pltpu.CompilerParams(
            dimension_semantics=("parallel",)),
    )(preds, targets[:_SPLIT].reshape(nb, r, 1))
    return g3.reshape(_SPLIT), nll3.reshape(_SPLIT)


def _bin_index(g, edg):
    b0 = jnp.clip((g * jnp.float32(_BINS)).astype(jnp.int32), 0, _BINS - 1)
    elo = plsc.load_gather(edg, [b0])
    ehi = plsc.load_gather(edg, [b0 + 1])
    b = b0 + (g >= ehi).astype(jnp.int32) - (g < elo).astype(jnp.int32)
    return jnp.clip(b, 0, _BINS - 1)


def _log_poly(svec):
    bits = plsc.bitcast(svec, jnp.int32)
    expo = (lax.shift_right_logical(bits, 23) & 0xFF) - 127
    mant = plsc.bitcast((bits & 0x7FFFFF) | 0x3F800000, jnp.float32)
    u = mant - jnp.float32(1.0)
    acc = jnp.full((16,), jnp.float32(_LOG2C[-1]), jnp.float32)
    for cc in _LOG2C[-2::-1]:
        acc = acc * u + jnp.float32(cc)
    return (expo.astype(jnp.float32) + acc) * jnp.float32(_LN2)


def _sc_rows_body(nch, rpw, preds_hbm, tgt_hbm, edges_hbm, parts_hbm,
                  b0, b1, tbuf, hist1, tmp, edg, sem, shall):
    cid = lax.axis_index("c")
    sid = lax.axis_index("s")
    wid = sid * 2 + cid
    row0 = _SPLIT + wid * rpw
    lane = lax.iota(jnp.int32, 16)
    ones = jnp.ones((16,), jnp.float32)
    zero16 = jnp.zeros((16,), jnp.float32)

    pltpu.sync_copy(tgt_hbm.at[pl.ds(row0, rpw)], tbuf)
    pltpu.sync_copy(edges_hbm, edg)
    for rr in range(64):
        hist1[pl.ds(rr * 16, 16)] = zero16

    pltpu.make_async_copy(preds_hbm.at[pl.ds(row0, _CHR)], b0, sem).start()
    pltpu.make_async_copy(preds_hbm.at[pl.ds(row0 + _CHR, _CHR)], b1, sem).start()

    def process_half(ch, buf, half):
        def row_body(j, carry):
            mvec, svec = carry
            r = half * 16 + j
            mt = [buf[r, pl.ds(k * 16, 16)] for k in range(4)]
            for c in range(4, 60, 4):
                for k in range(4):
                    mt[k] = jnp.maximum(mt[k], buf[r, pl.ds((c + k) * 16, 16)])
            mt[0] = jnp.maximum(mt[0], buf[r, pl.ds(60 * 16, 16)])
            mt[1] = jnp.maximum(mt[1], buf[r, pl.ds(61 * 16, 16)])
            m = jnp.max(jnp.maximum(jnp.maximum(mt[0], mt[1]),
                                    jnp.maximum(mt[2], mt[3])))
            mb = jnp.full((16,), m, jnp.float32)
            st = [jnp.exp(buf[r, pl.ds(k * 16, 16)] - mb) for k in range(4)]
            for c in range(4, 60, 4):
                for k in range(4):
                    st[k] = st[k] + jnp.exp(buf[r, pl.ds((c + k) * 16, 16)] - mb)
            st[0] = st[0] + jnp.exp(buf[r, pl.ds(60 * 16, 16)] - mb)
            st[1] = st[1] + jnp.exp(buf[r, pl.ds(61 * 16, 16)] - mb)
            s = jnp.sum((st[0] + st[1]) + (st[2] + st[3]))
            sel = lane == j
            mvec = jnp.where(sel, mb, mvec)
            svec = jnp.where(sel, jnp.full((16,), s, jnp.float32), svec)
            return (mvec, svec)

        mvec, svec = lax.fori_loop(0, 16, row_body, (zero16, zero16))

        rows16 = lane + (half * 16)
        tails = [plsc.load_gather(
            buf, [rows16, jnp.full((16,), 992 + k, jnp.int32)])
            for k in range(8)]
        tmax = tails[0]
        for tk in tails[1:]:
            tmax = jnp.maximum(tmax, tk)
        mnew = jnp.maximum(mvec, tmax)
        st = svec * jnp.exp(mvec - mnew)
        for tk in tails:
            st = st + jnp.exp(tk - mnew)

        t16 = tbuf[pl.ds(ch * _CHR + half * 16, 16)]
        xt = plsc.load_gather(buf, [rows16, t16])
        d = xt - mnew
        p = jnp.exp(d) / st
        pc = jnp.maximum(jnp.minimum(p, jnp.float32(1.0)), jnp.float32(_EPS))
        g = jnp.float32(1.0) - pc
        bidx = _bin_index(g, edg)
        nll = _log_poly(st) - d
        nll = jnp.where(p >= jnp.float32(1.0), jnp.float32(0.0), nll)
        nll = jnp.where(p <= jnp.float32(_EPS), jnp.float32(_NLL_EPS), nll)
        flat = bidx * 16 + lane
        plsc.addupdate_scatter(hist1, [flat], ones)
        plsc.addupdate_scatter(hist1, [flat + 512], nll)

    def super_body(g2, carry):
        for par, buf in ((0, b0), (1, b1)):
            ch = g2 * 2 + par
            pltpu.make_async_copy(
                preds_hbm.at[pl.ds(row0 + ch * _CHR, _CHR)], buf, sem).wait()
            process_half(ch, buf, 0)
            process_half(ch, buf, 1)

            @pl.when(ch + 2 < nch)
            def _():
                pltpu.make_async_copy(
                    preds_hbm.at[pl.ds(row0 + (ch + 2) * _CHR, _CHR)],
                    buf, sem).start()
        return carry

    lax.fori_loop(0, nch // 2, super_body, jnp.int32(0))

    pltpu.sync_copy(hist1, shall.at[pl.ds(sid * 1024, 1024)])
    plsc.subcore_barrier()

    @pl.when(sid == 0)
    def _():
        def acc_loop(w, c):
            pltpu.sync_copy(shall.at[pl.ds(w * 1024, 1024)], tmp)
            for rr in range(64):
                sl = pl.ds(rr * 16, 16)
                hist1[sl] = hist1[sl] + tmp[sl]
            return c

        lax.fori_loop(1, _NSUB, acc_loop, jnp.int32(0))
        pltpu.sync_copy(hist1, parts_hbm.at[pl.ds(cid * 1024, 1024)])


def _sc_rows(preds, targets):
    b = preds.shape[0]
    rpw = (b - _SPLIT) // 32
    nch = rpw // _CHR
    mesh = plsc.VectorSubcoreMesh(core_axis_name="c", subcore_axis_name="s")
    fn = functools.partial(
        pl.kernel,
        out_type=jax.ShapeDtypeStruct((2048,), jnp.float32),
        mesh=mesh,
        compiler_params=pltpu.CompilerParams(
            use_tc_tiling_on_sc=True, needs_layout_passes=False),
        scratch_types=[
            pltpu.VMEM((_CHR, 1000), jnp.float32),
            pltpu.VMEM((_CHR, 1000), jnp.float32),
            pltpu.VMEM((rpw,), jnp.int32),
            pltpu.VMEM((1024,), jnp.float32),
            pltpu.VMEM((1024,), jnp.float32),
            pltpu.VMEM((32,), jnp.float32),
            pltpu.SemaphoreType.DMA,
            pltpu.VMEM_SHARED((_NSUB * 1024,), jnp.float32),
        ],
    )(functools.partial(_sc_rows_body, nch, rpw))
    return fn(preds, targets, jnp.asarray(_edges_table()))


def _hist_body(chunk, g_hbm, nll_hbm, edges_hbm, parts_hbm, out_hbm,
               gbuf, nbuf, hist1, tmp, pbuf, edg, outv, shall):
    sid = lax.axis_index("s")
    base = sid * chunk
    pltpu.sync_copy(g_hbm.at[pl.ds(base, chunk)], gbuf)
    pltpu.sync_copy(nll_hbm.at[pl.ds(base, chunk)], nbuf)
    pltpu.sync_copy(edges_hbm, edg)

    zero16 = jnp.zeros((16,), jnp.float32)
    for rr in range(64):
        hist1[pl.ds(rr * 16, 16)] = zero16
    lane = lax.iota(jnp.int32, 16)
    ones = jnp.ones((16,), jnp.float32)

    def body(i, carry):
        off = i * 16
        g = gbuf[pl.ds(off, 16)]
        nll = nbuf[pl.ds(off, 16)]
        flat = _bin_index(g, edg) * 16 + lane
        plsc.addupdate_scatter(hist1, [flat], ones)
        plsc.addupdate_scatter(hist1, [flat + 512], nll)
        return carry

    lax.fori_loop(0, chunk // 16, body, jnp.int32(0))

    pltpu.sync_copy(hist1, shall.at[pl.ds(sid * 1024, 1024)])
    plsc.subcore_barrier()

    @pl.when(sid == 0)
    def _():
        def acc_loop(w, c):
            pltpu.sync_copy(shall.at[pl.ds(w * 1024, 1024)], tmp)
            for rr in range(64):
                sl = pl.ds(rr * 16, 16)
                hist1[sl] = hist1[sl] + tmp[sl]
            return c

        lax.fori_loop(1, _NSUB, acc_loop, jnp.int32(0))

        pltpu.sync_copy(parts_hbm, pbuf)
        for rr in range(64):
            sl = pl.ds(rr * 16, 16)
            hist1[sl] = (hist1[sl] + pbuf[pl.ds(rr * 16, 16)]
                         + pbuf[pl.ds(1024 + rr * 16, 16)])

        acc = jnp.zeros((16,), jnp.float32)
        n = jnp.float32(0.0)
        for bb in range(_BINS):
            cnt = jnp.sum(hist1[pl.ds(bb * 16, 16)])
            sb = jnp.sum(hist1[pl.ds(512 + bb * 16, 16)])
            acc = acc + (jnp.full((16,), sb, jnp.float32)
                         / jnp.full((16,), jnp.maximum(cnt, 1.0), jnp.float32))
            n = n + jnp.where(cnt > 0, jnp.float32(1.0), jnp.float32(0.0))
        loss_v = (jnp.float32(4.0) * acc) / jnp.full((16,), n, jnp.float32)
        outv[...] = loss_v
        pltpu.sync_copy(outv, out_hbm)


def _ghm_hist_loss(g, nll, parts):
    chunk = g.shape[0] // _NSUB
    mesh = plsc.VectorSubcoreMesh(
        core_axis_name="c", subcore_axis_name="s", num_cores=1)
    fn = functools.partial(
        pl.kernel,
        out_type=jax.ShapeDtypeStruct((16,), jnp.float32),
        mesh=mesh,
        compiler_params=pltpu.CompilerParams(needs_layout_passes=False),
        scratch_types=[
            pltpu.VMEM((chunk,), jnp.float32),
            pltpu.VMEM((chunk,), jnp.float32),
            pltpu.VMEM((1024,), jnp.float32),
            pltpu.VMEM((1024,), jnp.float32),
            pltpu.VMEM((2048,), jnp.float32),
            pltpu.VMEM((32,), jnp.float32),
            pltpu.VMEM((16,), jnp.float32),
            pltpu.VMEM_SHARED((_NSUB * 1024,), jnp.float32),
        ],
    )(functools.partial(_hist_body, chunk))
    out = fn(g, nll, jnp.asarray(_edges_table()), parts)
    return out[0]


def kernel(preds, targets):
    parts = _sc_rows(preds, targets)
    g, nll = _row_stats(preds, targets)
    return _ghm_hist_loss(g, nll, parts)


# speedup vs baseline: 1.1211x; 1.1211x over previous
import functools

import jax
import jax.numpy as jnp
import numpy as np
from jax import lax
from jax.experimental import pallas as pl
from jax.experimental.pallas import tpu as pltpu
from jax.experimental.pallas import tpu_sc as plsc

_BINS = 30
_EPS = 1e-10
_ROWS_PER_BLOCK = 2048
_NSUB = 16
_SPLIT = 32768
_CHR = 32
_LOG2C = (7.55886e-09, 1.4426935, -0.72129303, 0.4801555, -0.35541746,
          0.26644915, -0.18072152, 0.09595723, -0.03323721, 0.005413892)
_LN2 = 0.6931471805599453
_NLL_EPS = 23.02585


def _edges_table() -> np.ndarray:
    e = np.arange(32, dtype=np.float32) / np.float32(_BINS)
    e[30] = np.float32(1.0) + np.float32(_EPS)
    e[31] = np.inf
    return e


def _rows_body(preds_ref, tgt_ref, g_ref, nll_ref):
    x = preds_ref[...]
    t = tgt_ref[0]
    m = jnp.max(x, axis=1, keepdims=True)
    e = jnp.exp(x - m)
    s = jnp.sum(e, axis=1, keepdims=True)
    col = lax.broadcasted_iota(jnp.int32, x.shape, 1)
    et = jnp.sum(jnp.where(col == t, e, 0.0), axis=1, keepdims=True)
    p = et / s
    pc = jnp.clip(p, jnp.float32(_EPS), jnp.float32(1.0 - _EPS))
    g_ref[0] = jnp.abs(pc - 1.0)
    nll_ref[0] = -jnp.log(pc)


def _row_stats(preds, targets):
    c = preds.shape[1]
    r = _ROWS_PER_BLOCK
    nb = _SPLIT // r
    g3, nll3 = pl.pallas_call(
        _rows_body,
        grid=(nb,),
        in_specs=[
            pl.BlockSpec((r, c), lambda i: (i, 0)),
            pl.BlockSpec((1, r, 1), lambda i: (i, 0, 0)),
        ],
        out_specs=[
            pl.BlockSpec((1, r, 1), lambda i: (i, 0, 0)),
            pl.BlockSpec((1, r, 1), lambda i: (i, 0, 0)),
        ],
        out_shape=[jax.ShapeDtypeStruct((nb, r, 1), jnp.float32)] * 2,
        compiler_params=pltpu.CompilerParams(
            dimension_semantics=("parallel",)),
    )(preds, targets[:_SPLIT].reshape(nb, r, 1))
    return g3.reshape(_SPLIT), nll3.reshape(_SPLIT)


def _bin_index(g, edg):
    b0 = jnp.clip((g * jnp.float32(_BINS)).astype(jnp.int32), 0, _BINS - 1)
    elo = plsc.load_gather(edg, [b0])
    ehi = plsc.load_gather(edg, [b0 + 1])
    b = b0 + (g >= ehi).astype(jnp.int32) - (g < elo).astype(jnp.int32)
    return jnp.clip(b, 0, _BINS - 1)


def _log_poly(svec):
    bits = plsc.bitcast(svec, jnp.int32)
    expo = (lax.shift_right_logical(bits, 23) & 0xFF) - 127
    mant = plsc.bitcast((bits & 0x7FFFFF) | 0x3F800000, jnp.float32)
    u = mant - jnp.float32(1.0)
    acc = jnp.full((16,), jnp.float32(_LOG2C[-1]), jnp.float32)
    for cc in _LOG2C[-2::-1]:
        acc = acc * u + jnp.float32(cc)
    return (expo.astype(jnp.float32) + acc) * jnp.float32(_LN2)


def _sc_rows_body(nch, rpw, preds_hbm, tgt_hbm, edges_hbm, parts_hbm,
                  b0, b1, tbuf, hist1, tmp, edg, sem, shall):
    cid = lax.axis_index("c")
    sid = lax.axis_index("s")
    wid = sid * 2 + cid
    row0 = _SPLIT + wid * rpw
    lane = lax.iota(jnp.int32, 16)
    ones = jnp.ones((16,), jnp.float32)
    zero16 = jnp.zeros((16,), jnp.float32)

    pltpu.sync_copy(tgt_hbm.at[pl.ds(row0, rpw)], tbuf)
    pltpu.sync_copy(edges_hbm, edg)
    for rr in range(64):
        hist1[pl.ds(rr * 16, 16)] = zero16

    pltpu.make_async_copy(preds_hbm.at[pl.ds(row0, _CHR)], b0, sem).start()
    pltpu.make_async_copy(preds_hbm.at[pl.ds(row0 + _CHR, _CHR)], b1, sem).start()

    def process_half(ch, buf, half):
        def row_body(j, carry):
            mvec, svec = carry
            r = half * 16 + j
            mt = [buf[r, pl.ds(k * 16, 16)] for k in range(4)]
            for c in range(4, 60, 4):
                for k in range(4):
                    mt[k] = jnp.maximum(mt[k], buf[r, pl.ds((c + k) * 16, 16)])
            mt[0] = jnp.maximum(mt[0], buf[r, pl.ds(60 * 16, 16)])
            mt[1] = jnp.maximum(mt[1], buf[r, pl.ds(61 * 16, 16)])
            m = jnp.max(jnp.maximum(jnp.maximum(mt[0], mt[1]),
                                    jnp.maximum(mt[2], mt[3])))
            mb = jnp.full((16,), m, jnp.float32)
            st = [jnp.exp(buf[r, pl.ds(k * 16, 16)] - mb) for k in range(4)]
            for c in range(4, 60, 4):
                for k in range(4):
                    st[k] = st[k] + jnp.exp(buf[r, pl.ds((c + k) * 16, 16)] - mb)
            st[0] = st[0] + jnp.exp(buf[r, pl.ds(60 * 16, 16)] - mb)
            st[1] = st[1] + jnp.exp(buf[r, pl.ds(61 * 16, 16)] - mb)
            s = jnp.sum((st[0] + st[1]) + (st[2] + st[3]))
            sel = lane == j
            mvec = jnp.where(sel, mb, mvec)
            svec = jnp.where(sel, jnp.full((16,), s, jnp.float32), svec)
            return (mvec, svec)

        mvec, svec = lax.fori_loop(0, 16, row_body, (zero16, zero16))

        rows16 = lane + (half * 16)
        tails = [plsc.load_gather(
            buf, [rows16, jnp.full((16,), 992 + k, jnp.int32)])
            for k in range(8)]
        tmax = tails[0]
        for tk in tails[1:]:
            tmax = jnp.maximum(tmax, tk)
        mnew = jnp.maximum(mvec, tmax)
        st = svec * jnp.exp(mvec - mnew)
        for tk in tails:
            st = st + jnp.exp(tk - mnew)

        t16 = tbuf[pl.ds(ch * _CHR + half * 16, 16)]
        xt = plsc.load_gather(buf, [rows16, t16])
        d = xt - mnew
        p = jnp.exp(d) / st
        pc = jnp.maximum(jnp.minimum(p, jnp.float32(1.0)), jnp.float32(_EPS))
        g = jnp.float32(1.0) - pc
        bidx = _bin_index(g, edg)
        nll = _log_poly(st) - d
        nll = jnp.where(p >= jnp.float32(1.0), jnp.float32(0.0), nll)
        nll = jnp.where(p <= jnp.float32(_EPS), jnp.float32(_NLL_EPS), nll)
        flat = bidx * 16 + lane
        plsc.addupdate_scatter(hist1, [flat], ones)
        plsc.addupdate_scatter(hist1, [flat + 512], nll)

    def super_body(g2, carry):
        for par, buf in ((0, b0), (1, b1)):
            ch = g2 * 2 + par
            pltpu.make_async_copy(
                preds_hbm.at[pl.ds(row0 + ch * _CHR, _CHR)], buf, sem).wait()
            process_half(ch, buf, 0)
            process_half(ch, buf, 1)

            @pl.when(ch + 2 < nch)
            def _():
                pltpu.make_async_copy(
                    preds_hbm.at[pl.ds(row0 + (ch + 2) * _CHR, _CHR)],
                    buf, sem).start()
        return carry

    lax.fori_loop(0, nch // 2, super_body, jnp.int32(0))

    pltpu.sync_copy(hist1, shall.at[pl.ds(sid * 1024, 1024)])
    plsc.subcore_barrier()

    @pl.when(sid == 0)
    def _():
        def acc_loop(w, c):
            pltpu.sync_copy(shall.at[pl.ds(w * 1024, 1024)], tmp)
            for rr in range(64):
                sl = pl.ds(rr * 16, 16)
                hist1[sl] = hist1[sl] + tmp[sl]
            return c

        lax.fori_loop(1, _NSUB, acc_loop, jnp.int32(0))
        pltpu.sync_copy(hist1, parts_hbm.at[pl.ds(cid * 1024, 1024)])


def _sc_rows(preds, targets):
    b = preds.shape[0]
    rpw = (b - _SPLIT) // 32
    nch = rpw // _CHR
    mesh = plsc.VectorSubcoreMesh(core_axis_name="c", subcore_axis_name="s")
    fn = functools.partial(
        pl.kernel,
        out_type=jax.ShapeDtypeStruct((2048,), jnp.float32),
        mesh=mesh,
        compiler_params=pltpu.CompilerParams(
            use_tc_tiling_on_sc=True, needs_layout_passes=False),
        scratch_types=[
            pltpu.VMEM((_CHR, 1000), jnp.float32),
            pltpu.VMEM((_CHR, 1000), jnp.float32),
            pltpu.VMEM((rpw,), jnp.int32),
            pltpu.VMEM((1024,), jnp.float32),
            pltpu.VMEM((1024,), jnp.float32),
            pltpu.VMEM((32,), jnp.float32),
            pltpu.SemaphoreType.DMA,
            pltpu.VMEM_SHARED((_NSUB * 1024,), jnp.float32),
        ],
    )(functools.partial(_sc_rows_body, nch, rpw))
    return fn(preds, targets, jnp.asarray(_edges_table()))


def _hist_body(chunk, g_hbm, nll_hbm, edges_hbm, parts_hbm, out_hbm,
               gbuf, nbuf, hist1, tmp, pbuf, edg, outv, shall):
    sid = lax.axis_index("s")
    base = sid * chunk
    pltpu.sync_copy(g_hbm.at[pl.ds(base, chunk)], gbuf)
    pltpu.sync_copy(nll_hbm.at[pl.ds(base, chunk)], nbuf)
    pltpu.sync_copy(edges_hbm, edg)

    zero16 = jnp.zeros((16,), jnp.float32)
    for rr in range(64):
        hist1[pl.ds(rr * 16, 16)] = zero16
    lane = lax.iota(jnp.int32, 16)
    ones = jnp.ones((16,), jnp.float32)

    def body(i, carry):
        off = i * 16
        g = gbuf[pl.ds(off, 16)]
        nll = nbuf[pl.ds(off, 16)]
        flat = _bin_index(g, edg) * 16 + lane
        plsc.addupdate_scatter(hist1, [flat], ones)
        plsc.addupdate_scatter(hist1, [flat + 512], nll)
        return carry

    lax.fori_loop(0, chunk // 16, body, jnp.int32(0))

    pltpu.sync_copy(hist1, shall.at[pl.ds(sid * 1024, 1024)])
    plsc.subcore_barrier()

    @pl.when(sid == 0)
    def _():
        def acc_loop(w, c):
            pltpu.sync_copy(shall.at[pl.ds(w * 1024, 1024)], tmp)
            for rr in range(64):
                sl = pl.ds(rr * 16, 16)
                hist1[sl] = hist1[sl] + tmp[sl]
            return c

        lax.fori_loop(1, _NSUB, acc_loop, jnp.int32(0))

        pltpu.sync_copy(parts_hbm, pbuf)
        for rr in range(64):
            sl = pl.ds(rr * 16, 16)
            hist1[sl] = (hist1[sl] + pbuf[pl.ds(rr * 16, 16)]
                         + pbuf[pl.ds(1024 + rr * 16, 16)])

        acc = jnp.zeros((16,), jnp.float32)
        n = jnp.float32(0.0)
        for bb in range(_BINS):
            cnt = jnp.sum(hist1[pl.ds(bb * 16, 16)])
            sb = jnp.sum(hist1[pl.ds(512 + bb * 16, 16)])
            acc = acc + (jnp.full((16,), sb, jnp.float32)
                         / jnp.full((16,), jnp.maximum(cnt, 1.0), jnp.float32))
            n = n + jnp.where(cnt > 0, jnp.float32(1.0), jnp.float32(0.0))
        loss_v = (jnp.float32(4.0) * acc) / jnp.full((16,), n, jnp.float32)
        outv[...] = loss_v
        pltpu.sync_copy(outv, out_hbm)


def _ghm_hist_loss(g, nll, parts):
    chunk = g.shape[0] // _NSUB
    mesh = plsc.VectorSubcoreMesh(
        core_axis_name="c", subcore_axis_name="s", num_cores=1)
    fn = functools.partial(
        pl.kernel,
        out_type=jax.ShapeDtypeStruct((16,), jnp.float32),
        mesh=mesh,
        compiler_params=pltpu.CompilerParams(needs_layout_passes=False),
        scratch_types=[
            pltpu.VMEM((chunk,), jnp.float32),
            pltpu.VMEM((chunk,), jnp.float32),
            pltpu.VMEM((1024,), jnp.float32),
            pltpu.VMEM((1024,), jnp.float32),
            pltpu.VMEM((2048,), jnp.float32),
            pltpu.VMEM((32,), jnp.float32),
            pltpu.VMEM((16,), jnp.float32),
            pltpu.VMEM_SHARED((_NSUB * 1024,), jnp.float32),
        ],
    )(functools.partial(_hist_body, chunk))
    out = fn(g, nll, jnp.asarray(_edges_table()), parts)
    return out[0]


def kernel(preds, targets):
    parts = _sc_rows(preds, targets)
    g, nll = _row_stats(preds, targets)
    return _ghm_hist_loss(g, nll, parts)
